# baseline (speedup 1.0000x reference)
import functools

import jax
import jax.numpy as jnp
from jax import lax
from jax.experimental import pallas as pl
from jax.experimental.pallas import tpu as pltpu
from jax.experimental.pallas import tpu_sc as plsc

C = 64
NPTS = 2048
SX, SY, SZ = 100, 88, 80
VOL = SX * SY * SZ
NC, NS = 2, 16
NW = NC * NS
PPW = NPTS // NW
NV = PPW // 16
NROW = 32
LOSS_SCALE = 100.0 / (4.0 * NPTS)


def _sqrt16(x):
    i = plsc.bitcast(x, jnp.int32)
    y = plsc.bitcast(jnp.int32(0x5F3759DF) - (i >> 1), jnp.float32)
    for _ in range(3):
        y = y * (1.5 - 0.5 * x * y * y)
    return x * y


def _sc_body(fix_hbm, mov_hbm, pts_hbm,
             parts_hbm, pos_hbm, neg_hbm,
             pts_v, idx_v, rows_v, dis_v, part_v, red_v, shared, sem):
    cid = lax.axis_index("c")
    sid = lax.axis_index("s")
    wid = cid * NS + sid
    base = wid * PPW

    for r in range(9):
        pltpu.sync_copy(pts_hbm.at[pl.ds(r * NPTS + base, PPW)], pts_v.at[r])

    svecs = []
    for g in range(3):
        for i in range(NV):
            sl = pl.ds(i * 16, 16)
            x = lax.rem(pts_v[3 * g + 0, sl], SX)
            y = lax.rem(pts_v[3 * g + 1, sl], SY)
            z = lax.rem(pts_v[3 * g + 2, sl], SZ)
            svecs.append(x * (SY * SZ) + y * SZ + z)

    def build_row(j, sv):
        c0 = (2 * j) * VOL
        for g in range(3):
            row = idx_v.at[g, j]
            for half in range(2):
                coff = c0 + half * VOL
                for i in range(NV):
                    row[pl.ds(half * PPW + i * 16, 16)] = sv[NV * g + i] + coff
        return sv

    lax.fori_loop(0, NROW, build_row, tuple(svecs))

    def fire(j, _):
        pltpu.async_copy(fix_hbm.at[idx_v.at[0, j]], rows_v.at[0, j], sem)
        pltpu.async_copy(mov_hbm.at[idx_v.at[1, j]], rows_v.at[1, j], sem)
        pltpu.async_copy(mov_hbm.at[idx_v.at[2, j]], rows_v.at[2, j], sem)
        return 0

    lax.fori_loop(0, NROW, fire, 0)

    def drain(j, _):
        pltpu.make_async_copy(fix_hbm.at[idx_v.at[0, j]], rows_v.at[0, j], sem).wait()
        pltpu.make_async_copy(mov_hbm.at[idx_v.at[1, j]], rows_v.at[1, j], sem).wait()
        pltpu.make_async_copy(mov_hbm.at[idx_v.at[2, j]], rows_v.at[2, j], sem).wait()
        return 0

    lax.fori_loop(0, NROW, drain, 0)

    def accum(j, accs):
        accs = list(accs)
        for half in range(2):
            for i in range(NV):
                sl = pl.ds(half * PPW + i * 16, 16)
                a = rows_v[0, j, sl]
                p = rows_v[1, j, sl]
                n = rows_v[2, j, sl]
                dp = a - p
                dn = a - n
                accs[i] = accs[i] + dp * dp
                accs[NV + i] = accs[NV + i] + dn * dn
        return tuple(accs)

    zeros = jnp.zeros((16,), jnp.float32)
    accs = lax.fori_loop(0, NROW, accum, (zeros,) * (2 * NV))

    psum = zeros
    nsum = zeros
    for i in range(NV):
        pos_d2 = accs[i]
        neg_d2 = accs[NV + i]
        dis_v[pl.ds(i * 16, 16)] = _sqrt16(pos_d2)
        neg_dis = _sqrt16(neg_d2)
        dis_v[pl.ds(PPW + i * 16, 16)] = neg_dis
        hinge = jnp.maximum(1.0 - neg_dis, 0.0)
        psum = psum + pos_d2
        nsum = nsum + hinge * hinge

    pltpu.sync_copy(dis_v.at[pl.ds(0, PPW)], pos_hbm.at[pl.ds(base, PPW)])
    pltpu.sync_copy(dis_v.at[pl.ds(PPW, PPW)], neg_hbm.at[pl.ds(base, PPW)])

    lane = jnp.arange(16, dtype=jnp.int32)
    part_v[...] = jnp.where(lane == 0, jnp.sum((psum + nsum) * LOSS_SCALE), 0.0)
    pltpu.sync_copy(part_v.at[pl.ds(0, 8)], shared.at[pl.ds(sid * 8, 8)])
    plsc.subcore_barrier()

    @pl.when(sid == 0)
    def _():
        pltpu.sync_copy(shared, red_v)
        mask = lax.rem(lane, 8) == 0
        tot = jnp.zeros((16,), jnp.float32)
        for k in range(8):
            v = red_v[pl.ds(k * 16, 16)]
            tot = tot + jnp.where(mask, v, 0.0)
        part_v[...] = jnp.where(lane == 0, jnp.sum(tot), 0.0)
        pltpu.sync_copy(part_v.at[pl.ds(0, 8)], parts_hbm.at[pl.ds(cid * 8, 8)])


@jax.jit
def _sc_call(fix_flat, mov_flat, pts):
    mesh = plsc.VectorSubcoreMesh(core_axis_name="c", subcore_axis_name="s")
    run = functools.partial(
        pl.kernel,
        mesh=mesh,
        compiler_params=pltpu.CompilerParams(needs_layout_passes=False),
        out_type=[
            jax.ShapeDtypeStruct((NC * 8,), jnp.float32),
            jax.ShapeDtypeStruct((NPTS,), jnp.float32),
            jax.ShapeDtypeStruct((NPTS,), jnp.float32),
        ],
        scratch_types=[
            pltpu.VMEM((9, PPW), jnp.int32),
            pltpu.VMEM((3, NROW, 128), jnp.int32),
            pltpu.VMEM((3, NROW, 128), jnp.float32),
            pltpu.VMEM((2 * PPW,), jnp.float32),
            pltpu.VMEM((16,), jnp.float32),
            pltpu.VMEM((128,), jnp.float32),
            pltpu.VMEM_SHARED((128,), jnp.float32),
            pltpu.SemaphoreType.DMA,
        ],
    )(_sc_body)
    return run(fix_flat, mov_flat, pts)


def kernel(fix_image_feature, moving_image_feature, fixed_points,
           positive_points, negative_points, x_shard, y_shard, z_shard):
    fix_flat = fix_image_feature.reshape(-1)
    mov_flat = moving_image_feature.reshape(-1)
    pts = jnp.concatenate(
        [fixed_points.T, positive_points.T, negative_points.T], axis=0
    ).astype(jnp.int32).reshape(-1)
    parts, pos_dis, neg_dis = _sc_call(fix_flat, mov_flat, pts)
    loss = parts[0] + parts[8]
    return loss, pos_dis, neg_dis


# speedup vs baseline: 1.3231x; 1.3231x over previous
import functools

import jax
import jax.numpy as jnp
from jax import lax
from jax.experimental import pallas as pl
from jax.experimental.pallas import tpu as pltpu
from jax.experimental.pallas import tpu_sc as plsc

C = 64
NPTS = 2048
SX, SY, SZ = 100, 88, 80
VOL = SX * SY * SZ
NC, NS = 2, 16
NW = NC * NS
PPW = NPTS // NW
NV = PPW // 16
NROW = 32
LOSS_SCALE = 100.0 / (4.0 * NPTS)


def _sqrt16(x):
    i = plsc.bitcast(x, jnp.int32)
    y = plsc.bitcast(jnp.int32(0x5F3759DF) - (i >> 1), jnp.float32)
    for _ in range(3):
        y = y * (1.5 - 0.5 * x * y * y)
    return x * y


def _sc_body(fix_hbm, mov_hbm, pts_hbm,
             parts_hbm, pos_hbm, neg_hbm,
             pts_v, idx_v, rows_v, dis_v, part_v, red_v, shared, sem):
    cid = lax.axis_index("c")
    sid = lax.axis_index("s")
    wid = cid * NS + sid
    base = wid * PPW

    for r in range(9):
        pltpu.sync_copy(pts_hbm.at[pl.ds(r * NPTS + base, PPW)], pts_v.at[r])

    svecs = []
    for g in range(3):
        for i in range(NV):
            sl = pl.ds(i * 16, 16)
            x = lax.rem(pts_v[3 * g + 0, sl], SX)
            y = lax.rem(pts_v[3 * g + 1, sl], SY)
            z = lax.rem(pts_v[3 * g + 2, sl], SZ)
            svecs.append((y * SZ + z) * SX + x)

    def build_row(j, sv):
        c0 = (2 * j) * VOL
        for g in range(3):
            row = idx_v.at[g, j]
            for half in range(2):
                coff = c0 + half * VOL
                for i in range(NV):
                    row[pl.ds(half * PPW + i * 16, 16)] = sv[NV * g + i] + coff
        return sv

    lax.fori_loop(0, NROW, build_row, tuple(svecs))

    def fire(j, _):
        pltpu.async_copy(fix_hbm.at[idx_v.at[0, j]], rows_v.at[0, j], sem)
        pltpu.async_copy(mov_hbm.at[idx_v.at[1, j]], rows_v.at[1, j], sem)
        pltpu.async_copy(mov_hbm.at[idx_v.at[2, j]], rows_v.at[2, j], sem)
        return 0

    lax.fori_loop(0, NROW, fire, 0)

    def drain(j, _):
        pltpu.make_async_copy(fix_hbm.at[idx_v.at[0, j]], rows_v.at[0, j], sem).wait()
        pltpu.make_async_copy(mov_hbm.at[idx_v.at[1, j]], rows_v.at[1, j], sem).wait()
        pltpu.make_async_copy(mov_hbm.at[idx_v.at[2, j]], rows_v.at[2, j], sem).wait()
        return 0

    lax.fori_loop(0, NROW, drain, 0)

    def accum(j, accs):
        accs = list(accs)
        for half in range(2):
            for i in range(NV):
                sl = pl.ds(half * PPW + i * 16, 16)
                a = rows_v[0, j, sl]
                p = rows_v[1, j, sl]
                n = rows_v[2, j, sl]
                dp = a - p
                dn = a - n
                accs[i] = accs[i] + dp * dp
                accs[NV + i] = accs[NV + i] + dn * dn
        return tuple(accs)

    zeros = jnp.zeros((16,), jnp.float32)
    accs = lax.fori_loop(0, NROW, accum, (zeros,) * (2 * NV))

    psum = zeros
    nsum = zeros
    for i in range(NV):
        pos_d2 = accs[i]
        neg_d2 = accs[NV + i]
        dis_v[pl.ds(i * 16, 16)] = _sqrt16(pos_d2)
        neg_dis = _sqrt16(neg_d2)
        dis_v[pl.ds(PPW + i * 16, 16)] = neg_dis
        hinge = jnp.maximum(1.0 - neg_dis, 0.0)
        psum = psum + pos_d2
        nsum = nsum + hinge * hinge

    pltpu.sync_copy(dis_v.at[pl.ds(0, PPW)], pos_hbm.at[pl.ds(base, PPW)])
    pltpu.sync_copy(dis_v.at[pl.ds(PPW, PPW)], neg_hbm.at[pl.ds(base, PPW)])

    lane = jnp.arange(16, dtype=jnp.int32)
    part_v[...] = jnp.where(lane == 0, jnp.sum((psum + nsum) * LOSS_SCALE), 0.0)
    pltpu.sync_copy(part_v.at[pl.ds(0, 8)], shared.at[pl.ds(sid * 8, 8)])
    plsc.subcore_barrier()

    @pl.when(sid == 0)
    def _():
        pltpu.sync_copy(shared, red_v)
        mask = lax.rem(lane, 8) == 0
        tot = jnp.zeros((16,), jnp.float32)
        for k in range(8):
            v = red_v[pl.ds(k * 16, 16)]
            tot = tot + jnp.where(mask, v, 0.0)
        part_v[...] = jnp.where(lane == 0, jnp.sum(tot), 0.0)
        pltpu.sync_copy(part_v.at[pl.ds(0, 8)], parts_hbm.at[pl.ds(cid * 8, 8)])


@jax.jit
def _sc_call(fix_flat, mov_flat, pts):
    mesh = plsc.VectorSubcoreMesh(core_axis_name="c", subcore_axis_name="s")
    run = functools.partial(
        pl.kernel,
        mesh=mesh,
        compiler_params=pltpu.CompilerParams(needs_layout_passes=False),
        out_type=[
            jax.ShapeDtypeStruct((NC * 8,), jnp.float32),
            jax.ShapeDtypeStruct((NPTS,), jnp.float32),
            jax.ShapeDtypeStruct((NPTS,), jnp.float32),
        ],
        scratch_types=[
            pltpu.VMEM((9, PPW), jnp.int32),
            pltpu.VMEM((3, NROW, 128), jnp.int32),
            pltpu.VMEM((3, NROW, 128), jnp.float32),
            pltpu.VMEM((2 * PPW,), jnp.float32),
            pltpu.VMEM((16,), jnp.float32),
            pltpu.VMEM((128,), jnp.float32),
            pltpu.VMEM_SHARED((128,), jnp.float32),
            pltpu.SemaphoreType.DMA,
        ],
    )(_sc_body)
    return run(fix_flat, mov_flat, pts)


def kernel(fix_image_feature, moving_image_feature, fixed_points,
           positive_points, negative_points, x_shard, y_shard, z_shard):
    fix_flat = jnp.transpose(fix_image_feature, (0, 1, 3, 4, 2)).reshape(-1)
    mov_flat = jnp.transpose(moving_image_feature, (0, 1, 3, 4, 2)).reshape(-1)
    pts = jnp.concatenate(
        [fixed_points.T, positive_points.T, negative_points.T], axis=0
    ).astype(jnp.int32).reshape(-1)
    parts, pos_dis, neg_dis = _sc_call(fix_flat, mov_flat, pts)
    loss = parts[0] + parts[8]
    return loss, pos_dis, neg_dis
